# Initial kernel scaffold; baseline (speedup 1.0000x reference)
import jax
import jax.numpy as jnp
from jax.experimental import pallas as pl


def kernel(x_user, x_item, H_ui, H_iu, edge_index_ui, edge_index_iu):
    raise NotImplementedError("write your pallas kernel here")


# speedup vs baseline: 4.3281x; 4.3281x over previous
import functools

import jax
import jax.numpy as jnp
from jax import lax
from jax.experimental import pallas as pl
from jax.experimental.pallas import tpu as pltpu
from jax.experimental.pallas import tpu_sc as plsc

N = 10000
E = 160000
D = 128
LAM = 1.0
ALP = 1.0 / (LAM + 1.0)
CC = ALP * LAM
STEPS = 8

NC, NS = 2, 16
CH = 128
EPT = E // NS
NCHUNK = (EPT + CH - 1) // CH
EPAD = NCHUNK * CH
ACC_ROWS = 10112
ZPT = ACC_ROWS // NS
OPT = 624
TAIL = N - NS * OPT


def _mesh():
    return plsc.VectorSubcoreMesh(core_axis_name="c", subcore_axis_name="s",
                                  num_cores=NC, num_subcores=NS)


def _sc_gather_scatter_body(z_hbm, src_hbm, dst_hbm, g_hbm, src_v, dst_v,
                            rows_v, acc, sem):
    c = lax.axis_index("c")
    s = lax.axis_index("s")
    w = c * NS + s

    pltpu.sync_copy(src_hbm.at[w], src_v)
    pltpu.sync_copy(dst_hbm.at[w], dst_v)

    @pl.loop(0, CH)
    def _(r):
        for k in range(D // 16):
            rows_v[r, pl.ds(k * 16, 16)] = jnp.zeros((16,), jnp.float32)

    zbase = s * ZPT
    for k in range(ZPT // CH):
        pltpu.sync_copy(rows_v, acc.at[pl.ds(zbase + k * CH, CH)])
    rem = ZPT - (ZPT // CH) * CH
    if rem:
        pltpu.sync_copy(rows_v.at[pl.ds(0, rem)],
                        acc.at[pl.ds(zbase + (ZPT // CH) * CH, rem)])
    plsc.subcore_barrier()

    @pl.loop(0, NCHUNK)
    def _(j):
        pltpu.async_copy(z_hbm.at[src_v.at[j]], rows_v, sem).wait()
        pltpu.sync_copy(rows_v, acc.at[dst_v.at[j]], add=True)

    plsc.subcore_barrier()

    obase = s * OPT
    off = (1 - c) * N
    for k, sz in ((0, CH), (1, CH), (2, CH), (3, CH), (4, OPT - 4 * CH)):
        r0 = obase + k * CH
        pltpu.sync_copy(acc.at[pl.ds(r0, sz)], rows_v.at[pl.ds(0, sz)])
        pltpu.sync_copy(rows_v.at[pl.ds(0, sz)], g_hbm.at[pl.ds(off + r0, sz)])

    @pl.when(s == 0)
    def _():
        t0 = NS * OPT
        pltpu.sync_copy(acc.at[pl.ds(t0, TAIL)], rows_v.at[pl.ds(0, TAIL)])
        pltpu.sync_copy(rows_v.at[pl.ds(0, TAIL)],
                        g_hbm.at[pl.ds(off + t0, TAIL)])


@functools.lru_cache(maxsize=None)
def _sc_kernels():
    sc_gather_scatter = pl.kernel(
        _sc_gather_scatter_body,
        out_type=jax.ShapeDtypeStruct((NC * N, D), jnp.float32),
        mesh=_mesh(),
        scratch_types=[
            pltpu.VMEM((NCHUNK, CH), jnp.int32),
            pltpu.VMEM((NCHUNK, CH), jnp.int32),
            pltpu.VMEM((CH, D), jnp.float32),
            pltpu.VMEM_SHARED((ACC_ROWS, D), jnp.float32),
            pltpu.SemaphoreType.DMA,
        ],
    )
    return sc_gather_scatter


_RB = 1000
_NB = N // _RB


def _init_body(deg_ref, x_ref, inv_ref, z_ref):
    deg = deg_ref[0][:, 0:1]
    inv = lax.rsqrt(deg + 1.0)
    invb = jnp.broadcast_to(inv, (_RB, D))
    inv_ref[0] = invb
    z_ref[0] = x_ref[0] * invb


_tc_init = pl.pallas_call(
    _init_body,
    grid=(2, _NB),
    in_specs=[
        pl.BlockSpec((1, _RB, D), lambda n, b: (1 - n, b, 0)),
        pl.BlockSpec((1, _RB, D), lambda n, b: (n, b, 0)),
    ],
    out_specs=[
        pl.BlockSpec((1, _RB, D), lambda n, b: (n, b, 0)),
        pl.BlockSpec((1, _RB, D), lambda n, b: (n, b, 0)),
    ],
    out_shape=[
        jax.ShapeDtypeStruct((2, N, D), jnp.float32),
        jax.ShapeDtypeStruct((2, N, D), jnp.float32),
    ],
)


def _step_body(emit_y, g_ref, x_ref, inv_ref, h_ref, o_ref):
    agg = jnp.dot(g_ref[0], h_ref[0], preferred_element_type=jnp.float32,
                  precision=lax.Precision.HIGHEST)
    y = ALP * x_ref[0] + CC * inv_ref[0] * agg
    o_ref[0] = y if emit_y else y * inv_ref[0]


def _make_step(emit_y):
    return pl.pallas_call(
        functools.partial(_step_body, emit_y),
        grid=(2, _NB),
        in_specs=[
            pl.BlockSpec((1, _RB, D), lambda n, b: (n, b, 0)),
            pl.BlockSpec((1, _RB, D), lambda n, b: (n, b, 0)),
            pl.BlockSpec((1, _RB, D), lambda n, b: (n, b, 0)),
            pl.BlockSpec((1, D, D), lambda n, b: (n, 0, 0)),
        ],
        out_specs=pl.BlockSpec((1, _RB, D), lambda n, b: (n, b, 0)),
        out_shape=jax.ShapeDtypeStruct((2, N, D), jnp.float32),
    )


_tc_step_z = _make_step(False)
_tc_step_y = _make_step(True)


def _pack_edges(edge_index, etype):
    src = edge_index[0].reshape(NS, EPT)
    dst = edge_index[1].reshape(NS, EPT)
    srcg = jnp.full((NS, EPAD), etype * N, jnp.int32)
    srcg = srcg.at[:, :EPT].set(src + etype * N)
    dstp = jnp.full((NS, EPAD), N, jnp.int32).at[:, :EPT].set(dst)
    srcd = jnp.full((NS, EPAD), N, jnp.int32).at[:, :EPT].set(src)
    shp = (NS, NCHUNK, CH)
    return srcg.reshape(shp), dstp.reshape(shp), srcd.reshape(shp)


def kernel(x_user, x_item, H_ui, H_iu, edge_index_ui, edge_index_iu):
    sg0, dp0, sd0 = _pack_edges(edge_index_ui, 0)
    sg1, dp1, sd1 = _pack_edges(edge_index_iu, 1)
    srcs = jnp.concatenate([sg0, sg1], axis=0)
    dsts = jnp.concatenate([dp0, dp1], axis=0)
    sdeg = jnp.concatenate([sd0, sd1], axis=0)

    _sc_gather_scatter = _sc_kernels()
    ones_flat = jnp.ones((NC * N, D), jnp.float32)
    DEG = _sc_gather_scatter(ones_flat, srcs, sdeg)
    X = jnp.stack([x_user, x_item])
    INV, Z = _tc_init(DEG.reshape(2, N, D), X)
    HS = jnp.stack([H_iu, H_ui])

    for step in range(STEPS):
        G = _sc_gather_scatter(Z.reshape(NC * N, D), srcs, dsts)
        G = G.reshape(2, N, D)
        if step < STEPS - 1:
            Z = _tc_step_z(G, X, INV, HS)
        else:
            Y = _tc_step_y(G, X, INV, HS)
    return Y.reshape(NC * N, D)
